# baseline (speedup 1.0000x reference)
import jax
import jax.numpy as jnp
from jax import lax
from jax.experimental import pallas as pl
from jax.experimental.pallas import tpu as pltpu
from jax.experimental.pallas import tpu_sc as plsc

NC = 2
NS = 16
L = 16
CAPACITY = 1024


def kernel(token_hidden, expert_idx, slot_idx, expert_offsets):
    num_tokens, hidden = token_hidden.shape
    num_experts = expert_offsets.shape[0] - 1
    rows = num_experts * CAPACITY
    nw = NC * NS
    tpw = num_tokens // nw
    n_chunks = tpw // L
    cnt_iters = num_tokens // L
    zslots = CAPACITY // L // 2

    mesh = plsc.VectorSubcoreMesh(
        core_axis_name="c", subcore_axis_name="s", num_cores=NC, num_subcores=NS
    )

    def body(th_hbm, eidx_hbm, slot_hbm, off_hbm, out_hbm,
             eidx_v, eslice_v, slot_v, off_v, rows_v, zbuf_v,
             sidx_a, sidx_b, zidx_r, sem_in, sem_out, sem_z, sem_e):
        cid = lax.axis_index("c")
        sid = lax.axis_index("s")
        wid = sid * NC + cid
        base = wid * tpw

        in_cp = [pltpu.make_async_copy(
            th_hbm.at[pl.ds(base, L)], rows_v.at[0], sem_in)]
        in_cp[0].start()

        ecp = pltpu.make_async_copy(eidx_hbm, eidx_v, sem_e)
        ecp.start()
        pltpu.sync_copy(eidx_hbm.at[pl.ds(base, tpw)], eslice_v)
        pltpu.sync_copy(slot_hbm.at[pl.ds(base, tpw)], slot_v)
        pltpu.sync_copy(off_hbm.at[pl.ds(0, num_experts)], off_v)

        offv = off_v[...]
        e = wid // 2
        half = wid % 2
        iota = lax.iota(jnp.int32, L)

        def zinit(j, _):
            z = jnp.zeros((L,), jnp.float32)
            for r in range(L):
                zbuf_v[r, pl.ds(j * L, L)] = z
            return 0

        def cbody(i, acc):
            v = eidx_v[pl.ds(i * L, L)]
            return acc + jnp.where(v == e, 1, 0).astype(jnp.int32)

        k_cnt = cnt_iters // n_chunks
        k_z = (hidden // L) // n_chunks
        accv = jnp.zeros((L,), jnp.int32)
        sidx = (sidx_a, sidx_b)
        out_cp = []
        for c in range(n_chunks):
            b = c & 1
            in_cp[c].wait()
            ids = eslice_v[pl.ds(c * L, L)]
            offs = offv.at[ids].get(mode="promise_in_bounds")
            sidx[b][...] = offs + slot_v[pl.ds(c * L, L)]
            ocp = pltpu.make_async_copy(rows_v.at[b], out_hbm.at[sidx[b]], sem_out)
            ocp.start()
            out_cp.append(ocp)
            if c + 1 < n_chunks:
                if c >= 1:
                    out_cp[c - 1].wait()
                icp = pltpu.make_async_copy(
                    th_hbm.at[pl.ds(base + (c + 1) * L, L)],
                    rows_v.at[(c + 1) & 1], sem_in)
                icp.start()
                in_cp.append(icp)
            if c == 0:
                ecp.wait()
            accv = lax.fori_loop(c * k_cnt, (c + 1) * k_cnt, cbody, accv)
            lax.fori_loop(c * k_z, (c + 1) * k_z, zinit, 0)

        for k in (1, 2, 4, 8):
            accv = accv + accv.at[iota ^ k].get(mode="promise_in_bounds")
        cnt = accv[0]
        off_e = offv.at[jnp.where(iota == 0, e, iota)].get(
            mode="promise_in_bounds")[0]

        n_zero = CAPACITY - cnt
        aligned = (cnt + L - 1) // L * L
        nlin = (CAPACITY - aligned) // L

        @pl.when((half == 0) & (n_zero > 0))
        def _():
            p = jnp.minimum(cnt + iota, CAPACITY - 1)
            zidx_r[...] = off_e + p
            pltpu.make_async_copy(zbuf_v, out_hbm.at[zidx_r], sem_z).start()

        for k in range(zslots):
            g = 2 * k + half

            @pl.when(g < nlin)
            def _(g=g):
                zoff = pl.multiple_of(off_e + aligned + g * L, 8)
                pltpu.make_async_copy(
                    zbuf_v, out_hbm.at[pl.ds(zoff, L)], sem_z).start()

        my_issued = (jnp.maximum((nlin - half + 1) // 2, 0)
                     + jnp.where((half == 0) & (n_zero > 0), 1, 0))

        def dbody(i, _):
            pltpu.make_async_copy(
                th_hbm.at[pl.ds(0, L)], zbuf_v, sem_z).wait()
            return 0
        lax.fori_loop(0, my_issued, dbody, 0)

        if n_chunks >= 2:
            out_cp[n_chunks - 2].wait()
        out_cp[n_chunks - 1].wait()

    f = pl.kernel(
        body,
        out_type=jax.ShapeDtypeStruct((rows, hidden), token_hidden.dtype),
        mesh=mesh,
        scratch_types=[
            pltpu.VMEM((num_tokens,), jnp.int32),
            pltpu.VMEM((tpw,), jnp.int32),
            pltpu.VMEM((tpw,), jnp.int32),
            pltpu.VMEM((num_experts,), jnp.int32),
            pltpu.VMEM((2, L, hidden), jnp.float32),
            pltpu.VMEM((L, hidden), jnp.float32),
            pltpu.VMEM((L,), jnp.int32),
            pltpu.VMEM((L,), jnp.int32),
            pltpu.VMEM((L,), jnp.int32),
            pltpu.SemaphoreType.DMA,
            pltpu.SemaphoreType.DMA,
            pltpu.SemaphoreType.DMA,
            pltpu.SemaphoreType.DMA,
        ],
    )
    return f(token_hidden, expert_idx, slot_idx, expert_offsets)


# speedup vs baseline: 1.0258x; 1.0258x over previous
import jax
import jax.numpy as jnp
from jax import lax
from jax.experimental import pallas as pl
from jax.experimental.pallas import tpu as pltpu
from jax.experimental.pallas import tpu_sc as plsc

NC = 2
NS = 16
L = 16
CAPACITY = 1024


def kernel(token_hidden, expert_idx, slot_idx, expert_offsets):
    num_tokens, hidden = token_hidden.shape
    num_experts = expert_offsets.shape[0] - 1
    rows = num_experts * CAPACITY
    nw = NC * NS
    tpw = num_tokens // nw
    n_chunks = tpw // L
    cnt_iters = num_tokens // L
    zslots = CAPACITY // L // 2

    mesh = plsc.VectorSubcoreMesh(
        core_axis_name="c", subcore_axis_name="s", num_cores=NC, num_subcores=NS
    )

    def body(th_hbm, eidx_hbm, slot_hbm, off_hbm, out_hbm,
             eidx_v, eslice_v, slot_v, off_v, ridx_v, rows_v, zbuf_v,
             sidx_a, sidx_b, zidx_r, sem_in, sem_out, sem_z, sem_e):
        cid = lax.axis_index("c")
        sid = lax.axis_index("s")
        wid = sid * NC + cid
        base = wid * tpw

        in_cp = [pltpu.make_async_copy(
            th_hbm.at[pl.ds(base, L)], rows_v.at[0], sem_in)]
        in_cp[0].start()

        ecp = pltpu.make_async_copy(eidx_hbm, eidx_v, sem_e)
        ecp.start()
        pltpu.sync_copy(eidx_hbm.at[pl.ds(base, tpw)], eslice_v)
        pltpu.sync_copy(slot_hbm.at[pl.ds(base, tpw)], slot_v)
        pltpu.sync_copy(off_hbm.at[pl.ds(0, num_experts)], off_v)

        offv = off_v[...]
        for i in range(n_chunks):
            ids = eslice_v[pl.ds(i * L, L)]
            offs = offv.at[ids].get(mode="promise_in_bounds")
            ridx_v[pl.ds(i * L, L)] = offs + slot_v[pl.ds(i * L, L)]

        sidx = (sidx_a, sidx_b)
        out_cp = []
        for c in range(n_chunks):
            b = c & 1
            in_cp[c].wait()
            sidx[b][...] = ridx_v[pl.ds(c * L, L)]
            ocp = pltpu.make_async_copy(rows_v.at[b], out_hbm.at[sidx[b]], sem_out)
            ocp.start()
            out_cp.append(ocp)
            if c + 1 < n_chunks:
                if c >= 1:
                    out_cp[c - 1].wait()
                icp = pltpu.make_async_copy(
                    th_hbm.at[pl.ds(base + (c + 1) * L, L)],
                    rows_v.at[(c + 1) & 1], sem_in)
                icp.start()
                in_cp.append(icp)

        def zinit(j, _):
            z = jnp.zeros((L,), jnp.float32)
            for r in range(L):
                zbuf_v[r, pl.ds(j * L, L)] = z
            return 0
        lax.fori_loop(0, hidden // L, zinit, 0)

        e = wid // 2
        half = wid % 2
        iota = lax.iota(jnp.int32, L)
        ecp.wait()

        def cbody(i, acc):
            v = eidx_v[pl.ds(i * L, L)]
            return acc + jnp.where(v == e, 1, 0).astype(jnp.int32)
        accv = lax.fori_loop(
            0, cnt_iters, cbody, jnp.zeros((L,), jnp.int32))
        for k in (1, 2, 4, 8):
            accv = accv + accv.at[iota ^ k].get(mode="promise_in_bounds")
        cnt = accv[0]
        off_e = offv.at[jnp.where(iota == 0, e, iota)].get(
            mode="promise_in_bounds")[0]

        n_zero = CAPACITY - cnt
        aligned = (cnt + L - 1) // L * L
        nlin = (CAPACITY - aligned) // L

        @pl.when((half == 0) & (n_zero > 0))
        def _():
            p = jnp.minimum(cnt + iota, CAPACITY - 1)
            zidx_r[...] = off_e + p
            pltpu.make_async_copy(zbuf_v, out_hbm.at[zidx_r], sem_z).start()

        for k in range(zslots):
            g = 2 * k + half

            @pl.when(g < nlin)
            def _(g=g):
                zoff = pl.multiple_of(off_e + aligned + g * L, 8)
                pltpu.make_async_copy(
                    zbuf_v, out_hbm.at[pl.ds(zoff, L)], sem_z).start()

        my_issued = (jnp.maximum((nlin - half + 1) // 2, 0)
                     + jnp.where((half == 0) & (n_zero > 0), 1, 0))

        def dbody(i, _):
            pltpu.make_async_copy(
                th_hbm.at[pl.ds(0, L)], zbuf_v, sem_z).wait()
            return 0
        lax.fori_loop(0, my_issued, dbody, 0)

        if n_chunks >= 2:
            out_cp[n_chunks - 2].wait()
        out_cp[n_chunks - 1].wait()

    f = pl.kernel(
        body,
        out_type=jax.ShapeDtypeStruct((rows, hidden), token_hidden.dtype),
        mesh=mesh,
        scratch_types=[
            pltpu.VMEM((num_tokens,), jnp.int32),
            pltpu.VMEM((tpw,), jnp.int32),
            pltpu.VMEM((tpw,), jnp.int32),
            pltpu.VMEM((num_experts,), jnp.int32),
            pltpu.VMEM((tpw,), jnp.int32),
            pltpu.VMEM((2, L, hidden), jnp.float32),
            pltpu.VMEM((L, hidden), jnp.float32),
            pltpu.VMEM((L,), jnp.int32),
            pltpu.VMEM((L,), jnp.int32),
            pltpu.VMEM((L,), jnp.int32),
            pltpu.SemaphoreType.DMA,
            pltpu.SemaphoreType.DMA,
            pltpu.SemaphoreType.DMA,
            pltpu.SemaphoreType.DMA,
        ],
    )
    return f(token_hidden, expert_idx, slot_idx, expert_offsets)


# speedup vs baseline: 1.0312x; 1.0053x over previous
import jax
import jax.numpy as jnp
from jax import lax
from jax.experimental import pallas as pl
from jax.experimental.pallas import tpu as pltpu
from jax.experimental.pallas import tpu_sc as plsc

NC = 2
NS = 16
L = 16
CAPACITY = 1024


def kernel(token_hidden, expert_idx, slot_idx, expert_offsets):
    num_tokens, hidden = token_hidden.shape
    num_experts = expert_offsets.shape[0] - 1
    rows = num_experts * CAPACITY
    nw = NC * NS
    tpw = num_tokens // nw
    n_chunks = tpw // L
    cnt_iters = num_tokens // L
    zslots = CAPACITY // L // 2

    mesh = plsc.VectorSubcoreMesh(
        core_axis_name="c", subcore_axis_name="s", num_cores=NC, num_subcores=NS
    )

    def body(th_hbm, eidx_hbm, slot_hbm, off_hbm, out_hbm,
             eidx_v, eslice_v, slot_v, off_v, rows_v, zbuf_v,
             sidx_a, sidx_b, zidx_r, sem_in, sem_out, sem_z, sem_e):
        cid = lax.axis_index("c")
        sid = lax.axis_index("s")
        wid = sid * NC + cid
        base = wid * tpw

        in_cp = [pltpu.make_async_copy(
            th_hbm.at[pl.ds(base, L)], rows_v.at[0], sem_in)]
        in_cp[0].start()

        ecp = pltpu.make_async_copy(eidx_hbm, eidx_v, sem_e)
        ecp.start()
        pltpu.sync_copy(eidx_hbm.at[pl.ds(base, tpw)], eslice_v)
        pltpu.sync_copy(slot_hbm.at[pl.ds(base, tpw)], slot_v)
        pltpu.sync_copy(off_hbm.at[pl.ds(0, num_experts)], off_v)

        offv = off_v[...]
        sidx = (sidx_a, sidx_b)
        out_cp = []
        for c in range(n_chunks):
            b = c & 1
            in_cp[c].wait()
            ids = eslice_v[pl.ds(c * L, L)]
            offs = offv.at[ids].get(mode="promise_in_bounds")
            sidx[b][...] = offs + slot_v[pl.ds(c * L, L)]
            ocp = pltpu.make_async_copy(rows_v.at[b], out_hbm.at[sidx[b]], sem_out)
            ocp.start()
            out_cp.append(ocp)
            if c + 1 < n_chunks:
                if c >= 1:
                    out_cp[c - 1].wait()
                icp = pltpu.make_async_copy(
                    th_hbm.at[pl.ds(base + (c + 1) * L, L)],
                    rows_v.at[(c + 1) & 1], sem_in)
                icp.start()
                in_cp.append(icp)

        def zinit(j, _):
            z = jnp.zeros((L,), jnp.float32)
            for r in range(L):
                zbuf_v[r, pl.ds(j * L, L)] = z
            return 0
        lax.fori_loop(0, hidden // L, zinit, 0)

        e = wid // 2
        half = wid % 2
        iota = lax.iota(jnp.int32, L)
        ecp.wait()

        def cbody(i, acc):
            v = eidx_v[pl.ds(i * L, L)]
            return acc + jnp.where(v == e, 1, 0).astype(jnp.int32)
        accv = lax.fori_loop(
            0, cnt_iters, cbody, jnp.zeros((L,), jnp.int32))
        for k in (1, 2, 4, 8):
            accv = accv + accv.at[iota ^ k].get(mode="promise_in_bounds")
        cnt = accv[0]
        off_e = offv.at[jnp.where(iota == 0, e, iota)].get(
            mode="promise_in_bounds")[0]

        n_zero = CAPACITY - cnt
        aligned = (cnt + L - 1) // L * L
        nlin = (CAPACITY - aligned) // L

        @pl.when((half == 0) & (n_zero > 0))
        def _():
            p = jnp.minimum(cnt + iota, CAPACITY - 1)
            zidx_r[...] = off_e + p
            pltpu.make_async_copy(zbuf_v, out_hbm.at[zidx_r], sem_z).start()

        for k in range(zslots):
            g = 2 * k + half

            @pl.when(g < nlin)
            def _(g=g):
                zoff = pl.multiple_of(off_e + aligned + g * L, 8)
                pltpu.make_async_copy(
                    zbuf_v, out_hbm.at[pl.ds(zoff, L)], sem_z).start()

        my_issued = (jnp.maximum((nlin - half + 1) // 2, 0)
                     + jnp.where((half == 0) & (n_zero > 0), 1, 0))

        def dbody(i, _):
            pltpu.make_async_copy(
                th_hbm.at[pl.ds(0, L)], zbuf_v, sem_z).wait()
            return 0
        lax.fori_loop(0, my_issued, dbody, 0)

        if n_chunks >= 2:
            out_cp[n_chunks - 2].wait()
        out_cp[n_chunks - 1].wait()

    f = pl.kernel(
        body,
        out_type=jax.ShapeDtypeStruct((rows, hidden), token_hidden.dtype),
        mesh=mesh,
        scratch_types=[
            pltpu.VMEM((num_tokens,), jnp.int32),
            pltpu.VMEM((tpw,), jnp.int32),
            pltpu.VMEM((tpw,), jnp.int32),
            pltpu.VMEM((num_experts,), jnp.int32),
            pltpu.VMEM((2, L, hidden), jnp.float32),
            pltpu.VMEM((L, hidden), jnp.float32),
            pltpu.VMEM((L,), jnp.int32),
            pltpu.VMEM((L,), jnp.int32),
            pltpu.VMEM((L,), jnp.int32),
            pltpu.SemaphoreType.DMA,
            pltpu.SemaphoreType.DMA,
            pltpu.SemaphoreType.DMA,
            pltpu.SemaphoreType.DMA,
        ],
    )
    return f(token_hidden, expert_idx, slot_idx, expert_offsets)
